# baseline (speedup 1.0000x reference)
import jax
import jax.numpy as jnp
from jax.experimental import pallas as pl
from jax.experimental.pallas import tpu as pltpu

HID = 128
NT = 9
TS = [3, 1, 3, 1]
RES = {1: [0], 3: [0, 1]}
BLK = 256


def _segmm_body(tm_ref, x_ref, w_ref, b_ref, o_ref):
    del tm_ref
    o_ref[...] = (
        jnp.dot(x_ref[...], w_ref[0], preferred_element_type=jnp.float32)
        + b_ref[0]
    )


def _segmm(gathered, w, bias, tmap):
    cap = gathered.shape[0]
    nb = cap // BLK
    grid_spec = pltpu.PrefetchScalarGridSpec(
        num_scalar_prefetch=1,
        grid=(nb,),
        in_specs=[
            pl.BlockSpec((BLK, HID), lambda j, tm: (j, 0)),
            pl.BlockSpec((1, HID, HID), lambda j, tm: (tm[j], 0, 0)),
            pl.BlockSpec((1, 1, HID), lambda j, tm: (tm[j], 0, 0)),
        ],
        out_specs=pl.BlockSpec((BLK, HID), lambda j, tm: (j, 0)),
    )
    return pl.pallas_call(
        _segmm_body,
        grid_spec=grid_spec,
        out_shape=jax.ShapeDtypeStruct((cap, HID), jnp.float32),
    )(tmap, gathered, w, bias.reshape(NT, 1, HID))


def _gru_body(x_ref, h_ref, k_ref, rk_ref, b0_ref, b1_ref, o_ref):
    mx = (
        jnp.dot(x_ref[...], k_ref[...], preferred_element_type=jnp.float32)
        + b0_ref[...]
    )
    mh = (
        jnp.dot(h_ref[...], rk_ref[...], preferred_element_type=jnp.float32)
        + b1_ref[...]
    )
    h = h_ref[...]
    z = jax.nn.sigmoid(mx[:, :HID] + mh[:, :HID])
    r = jax.nn.sigmoid(mx[:, HID:2 * HID] + mh[:, HID:2 * HID])
    hh = jnp.tanh(mx[:, 2 * HID:] + r * mh[:, 2 * HID:])
    o_ref[...] = z * h + (1.0 - z) * hh


def _gru(xcat, h, k, rk, b0, b1, rb=2000):
    r, d = xcat.shape
    return pl.pallas_call(
        _gru_body,
        grid=(r // rb,),
        in_specs=[
            pl.BlockSpec((rb, d), lambda i: (i, 0)),
            pl.BlockSpec((rb, HID), lambda i: (i, 0)),
            pl.BlockSpec((d, 3 * HID), lambda i: (0, 0)),
            pl.BlockSpec((HID, 3 * HID), lambda i: (0, 0)),
            pl.BlockSpec((1, 3 * HID), lambda i: (0, 0)),
            pl.BlockSpec((1, 3 * HID), lambda i: (0, 0)),
        ],
        out_specs=pl.BlockSpec((rb, HID), lambda i: (i, 0)),
        out_shape=jax.ShapeDtypeStruct((r, HID), jnp.float32),
    )(xcat, h, k, rk, b0, b1)


def kernel(states, edge_ids, Wt, bt, gru_k0, gru_rk0, gru_b0, gru_k1, gru_rk1,
           gru_b1, gru_k2, gru_rk2, gru_b2, gru_k3, gru_rk3, gru_b3):
    gk = [gru_k0, gru_k1, gru_k2, gru_k3]
    grk = [gru_rk0, gru_rk1, gru_rk2, gru_rk3]
    gb = [gru_b0, gru_b1, gru_b2, gru_b3]
    b, n, h_dim = states.shape
    bn = b * n
    e = edge_ids.shape[0]
    etype = edge_ids[:, 0]
    eb = edge_ids[:, 1]
    es = edge_ids[:, 2]
    ed = edge_ids[:, 3]

    cap = ((e + NT * BLK) // BLK) * BLK
    order = jnp.argsort(etype)
    ts = etype[order]
    tgrid = jnp.arange(NT, dtype=jnp.int32)
    starts = jnp.searchsorted(ts, tgrid, side="left").astype(jnp.int32)
    ends = jnp.searchsorted(ts, tgrid, side="right").astype(jnp.int32)
    pc = ((ends - starts + BLK - 1) // BLK) * BLK
    pstart = jnp.concatenate(
        [jnp.zeros((1,), jnp.int32), jnp.cumsum(pc).astype(jnp.int32)]
    )
    pos = pstart[ts] + (jnp.arange(e, dtype=jnp.int32) - starts[ts])
    gsrc_pad = jnp.zeros((cap,), jnp.int32).at[pos].set((eb * n + es)[order])
    gdst_pad = jnp.full((cap,), bn, jnp.int32).at[pos].set((eb * n + ed)[order])
    offs = jnp.arange(cap // BLK, dtype=jnp.int32) * BLK
    tmap = jnp.clip(
        jnp.searchsorted(pstart, offs, side="right") - 1, 0, NT - 1
    ).astype(jnp.int32)

    layer_states = [states.reshape(bn, h_dim)]
    for l, steps in enumerate(TS):
        k, rk = gk[l], grk[l]
        b0, b1 = gb[l][0:1], gb[l][1:2]
        for s in range(steps):
            h = layer_states[-1]
            gathered = jnp.take(h, gsrc_pad, axis=0)
            m = _segmm(gathered, Wt[l], bt[l], tmap)
            msgs = jnp.zeros((bn + 8, h_dim), jnp.float32).at[gdst_pad].add(m)[:bn]
            parts = [layer_states[ix] for ix in RES.get(l, [])] + [msgs]
            xcat = jnp.concatenate(parts, axis=1) if len(parts) > 1 else msgs
            new = _gru(xcat, h, k, rk, b0, b1)
            if s == 0:
                layer_states.append(new)
            else:
                layer_states[-1] = new
    return layer_states[-1].reshape(b, n, h_dim)


# speedup vs baseline: 1.0213x; 1.0213x over previous
import jax
import jax.numpy as jnp
from jax.experimental import pallas as pl
from jax.experimental.pallas import tpu as pltpu

HID = 128
NT = 9
TS = [3, 1, 3, 1]
RES = {1: [0], 3: [0, 1]}
BLK = 256


def _segmm_body(tm_ref, x_ref, w_ref, b_ref, o_ref):
    del tm_ref
    o_ref[...] = (
        jnp.dot(x_ref[...], w_ref[0], preferred_element_type=jnp.float32)
        + b_ref[0]
    )


def _segmm(gathered, w, bias, tmap):
    cap = gathered.shape[0]
    nb = cap // BLK
    grid_spec = pltpu.PrefetchScalarGridSpec(
        num_scalar_prefetch=1,
        grid=(nb,),
        in_specs=[
            pl.BlockSpec((BLK, HID), lambda j, tm: (j, 0)),
            pl.BlockSpec((1, HID, HID), lambda j, tm: (tm[j], 0, 0)),
            pl.BlockSpec((1, 1, HID), lambda j, tm: (tm[j], 0, 0)),
        ],
        out_specs=pl.BlockSpec((BLK, HID), lambda j, tm: (j, 0)),
    )
    return pl.pallas_call(
        _segmm_body,
        grid_spec=grid_spec,
        out_shape=jax.ShapeDtypeStruct((cap, HID), jnp.float32),
    )(tmap, gathered, w, bias.reshape(NT, 1, HID))


def _gru_body(x_ref, h_ref, k_ref, rk_ref, b0_ref, b1_ref, o_ref):
    mx = (
        jnp.dot(x_ref[...], k_ref[...], preferred_element_type=jnp.float32)
        + b0_ref[...]
    )
    mh = (
        jnp.dot(h_ref[...], rk_ref[...], preferred_element_type=jnp.float32)
        + b1_ref[...]
    )
    h = h_ref[...]
    z = jax.nn.sigmoid(mx[:, :HID] + mh[:, :HID])
    r = jax.nn.sigmoid(mx[:, HID:2 * HID] + mh[:, HID:2 * HID])
    hh = jnp.tanh(mx[:, 2 * HID:] + r * mh[:, 2 * HID:])
    o_ref[...] = z * h + (1.0 - z) * hh


def _gru(xcat, h, k, rk, b0, b1, rb=2000):
    r, d = xcat.shape
    return pl.pallas_call(
        _gru_body,
        grid=(r // rb,),
        in_specs=[
            pl.BlockSpec((rb, d), lambda i: (i, 0)),
            pl.BlockSpec((rb, HID), lambda i: (i, 0)),
            pl.BlockSpec((d, 3 * HID), lambda i: (0, 0)),
            pl.BlockSpec((HID, 3 * HID), lambda i: (0, 0)),
            pl.BlockSpec((1, 3 * HID), lambda i: (0, 0)),
            pl.BlockSpec((1, 3 * HID), lambda i: (0, 0)),
        ],
        out_specs=pl.BlockSpec((rb, HID), lambda i: (i, 0)),
        out_shape=jax.ShapeDtypeStruct((r, HID), jnp.float32),
    )(xcat, h, k, rk, b0, b1)


def kernel(states, edge_ids, Wt, bt, gru_k0, gru_rk0, gru_b0, gru_k1, gru_rk1,
           gru_b1, gru_k2, gru_rk2, gru_b2, gru_k3, gru_rk3, gru_b3):
    gk = [gru_k0, gru_k1, gru_k2, gru_k3]
    grk = [gru_rk0, gru_rk1, gru_rk2, gru_rk3]
    gb = [gru_b0, gru_b1, gru_b2, gru_b3]
    b, n, h_dim = states.shape
    bn = b * n
    e = edge_ids.shape[0]
    etype = edge_ids[:, 0]
    eb = edge_ids[:, 1]
    es = edge_ids[:, 2]
    ed = edge_ids[:, 3]

    cap = ((e + NT * BLK) // BLK) * BLK
    tgrid = jnp.arange(NT, dtype=jnp.int32)
    onehot = (etype[None, :] == tgrid[:, None]).astype(jnp.int32)
    occ = jnp.cumsum(onehot, axis=1)
    cnts = occ[:, -1]
    pc = ((cnts + BLK - 1) // BLK) * BLK
    pstart = jnp.concatenate(
        [jnp.zeros((1,), jnp.int32), jnp.cumsum(pc).astype(jnp.int32)]
    )
    pos = jnp.sum(onehot * (pstart[:NT, None] + occ - 1), axis=0)
    gsrc_pad = jnp.zeros((cap,), jnp.int32).at[pos].set(eb * n + es)
    gdst_pad = jnp.full((cap,), bn, jnp.int32).at[pos].set(eb * n + ed)
    offs = jnp.arange(cap // BLK, dtype=jnp.int32) * BLK
    tmap = jnp.clip(
        jnp.searchsorted(pstart, offs, side="right") - 1, 0, NT - 1
    ).astype(jnp.int32)

    layer_states = [states.reshape(bn, h_dim)]
    for l, steps in enumerate(TS):
        k, rk = gk[l], grk[l]
        b0, b1 = gb[l][0:1], gb[l][1:2]
        for s in range(steps):
            h = layer_states[-1]
            gathered = jnp.take(h, gsrc_pad, axis=0)
            m = _segmm(gathered, Wt[l], bt[l], tmap)
            msgs = jnp.zeros((bn + 8, h_dim), jnp.float32).at[gdst_pad].add(m)[:bn]
            parts = [layer_states[ix] for ix in RES.get(l, [])] + [msgs]
            xcat = jnp.concatenate(parts, axis=1) if len(parts) > 1 else msgs
            new = _gru(xcat, h, k, rk, b0, b1)
            if s == 0:
                layer_states.append(new)
            else:
                layer_states[-1] = new
    return layer_states[-1].reshape(b, n, h_dim)


# speedup vs baseline: 1.0386x; 1.0170x over previous
import jax
import jax.numpy as jnp
from jax.experimental import pallas as pl
from jax.experimental.pallas import tpu as pltpu

HID = 128
NT = 9
TS = [3, 1, 3, 1]
RES = {1: [0], 3: [0, 1]}
BLK = 1024


def _segmm_body(tm_ref, x_ref, w_ref, b_ref, o_ref):
    del tm_ref
    o_ref[...] = (
        jnp.dot(x_ref[...], w_ref[0], preferred_element_type=jnp.float32)
        + b_ref[0]
    )


def _segmm(gathered, w, bias, tmap):
    cap = gathered.shape[0]
    nb = cap // BLK
    grid_spec = pltpu.PrefetchScalarGridSpec(
        num_scalar_prefetch=1,
        grid=(nb,),
        in_specs=[
            pl.BlockSpec((BLK, HID), lambda j, tm: (j, 0)),
            pl.BlockSpec((1, HID, HID), lambda j, tm: (tm[j], 0, 0)),
            pl.BlockSpec((1, 1, HID), lambda j, tm: (tm[j], 0, 0)),
        ],
        out_specs=pl.BlockSpec((BLK, HID), lambda j, tm: (j, 0)),
    )
    return pl.pallas_call(
        _segmm_body,
        grid_spec=grid_spec,
        out_shape=jax.ShapeDtypeStruct((cap, HID), jnp.float32),
    )(tmap, gathered, w, bias.reshape(NT, 1, HID))


def _gru_body(x_ref, h_ref, k_ref, rk_ref, b0_ref, b1_ref, o_ref):
    mx = (
        jnp.dot(x_ref[...], k_ref[...], preferred_element_type=jnp.float32)
        + b0_ref[...]
    )
    mh = (
        jnp.dot(h_ref[...], rk_ref[...], preferred_element_type=jnp.float32)
        + b1_ref[...]
    )
    h = h_ref[...]
    z = jax.nn.sigmoid(mx[:, :HID] + mh[:, :HID])
    r = jax.nn.sigmoid(mx[:, HID:2 * HID] + mh[:, HID:2 * HID])
    hh = jnp.tanh(mx[:, 2 * HID:] + r * mh[:, 2 * HID:])
    o_ref[...] = z * h + (1.0 - z) * hh


def _gru(xcat, h, k, rk, b0, b1, rb=2000):
    r, d = xcat.shape
    return pl.pallas_call(
        _gru_body,
        grid=(r // rb,),
        in_specs=[
            pl.BlockSpec((rb, d), lambda i: (i, 0)),
            pl.BlockSpec((rb, HID), lambda i: (i, 0)),
            pl.BlockSpec((d, 3 * HID), lambda i: (0, 0)),
            pl.BlockSpec((HID, 3 * HID), lambda i: (0, 0)),
            pl.BlockSpec((1, 3 * HID), lambda i: (0, 0)),
            pl.BlockSpec((1, 3 * HID), lambda i: (0, 0)),
        ],
        out_specs=pl.BlockSpec((rb, HID), lambda i: (i, 0)),
        out_shape=jax.ShapeDtypeStruct((r, HID), jnp.float32),
    )(xcat, h, k, rk, b0, b1)


def kernel(states, edge_ids, Wt, bt, gru_k0, gru_rk0, gru_b0, gru_k1, gru_rk1,
           gru_b1, gru_k2, gru_rk2, gru_b2, gru_k3, gru_rk3, gru_b3):
    gk = [gru_k0, gru_k1, gru_k2, gru_k3]
    grk = [gru_rk0, gru_rk1, gru_rk2, gru_rk3]
    gb = [gru_b0, gru_b1, gru_b2, gru_b3]
    b, n, h_dim = states.shape
    bn = b * n
    e = edge_ids.shape[0]
    etype = edge_ids[:, 0]
    eb = edge_ids[:, 1]
    es = edge_ids[:, 2]
    ed = edge_ids[:, 3]

    cap = ((e + NT * (BLK - 1) + BLK - 1) // BLK) * BLK
    tgrid = jnp.arange(NT, dtype=jnp.int32)
    onehot = (etype[None, :] == tgrid[:, None]).astype(jnp.int32)
    occ = jnp.cumsum(onehot, axis=1)
    cnts = occ[:, -1]
    pc = ((cnts + BLK - 1) // BLK) * BLK
    pstart = jnp.concatenate(
        [jnp.zeros((1,), jnp.int32), jnp.cumsum(pc).astype(jnp.int32)]
    )
    pos = jnp.sum(onehot * (pstart[:NT, None] + occ - 1), axis=0)
    eid_pad = jnp.full((cap,), e, jnp.int32).at[pos].set(
        jnp.arange(e, dtype=jnp.int32)
    )
    gsrc_pad = jnp.concatenate([eb * n + es, jnp.zeros((1,), jnp.int32)])[eid_pad]
    gdst_pad = jnp.concatenate([eb * n + ed, jnp.full((1,), bn, jnp.int32)])[eid_pad]
    offs = jnp.arange(cap // BLK, dtype=jnp.int32) * BLK
    tmap = jnp.clip(
        jnp.searchsorted(pstart, offs, side="right") - 1, 0, NT - 1
    ).astype(jnp.int32)

    layer_states = [states.reshape(bn, h_dim)]
    for l, steps in enumerate(TS):
        k, rk = gk[l], grk[l]
        b0, b1 = gb[l][0:1], gb[l][1:2]
        for s in range(steps):
            h = layer_states[-1]
            gathered = jnp.take(h, gsrc_pad, axis=0)
            m = _segmm(gathered, Wt[l], bt[l], tmap)
            msgs = jnp.zeros((bn + 8, h_dim), jnp.float32).at[gdst_pad].add(m)[:bn]
            parts = [layer_states[ix] for ix in RES.get(l, [])] + [msgs]
            xcat = jnp.concatenate(parts, axis=1) if len(parts) > 1 else msgs
            new = _gru(xcat, h, k, rk, b0, b1)
            if s == 0:
                layer_states.append(new)
            else:
                layer_states[-1] = new
    return layer_states[-1].reshape(b, n, h_dim)
